# baseline (device time: 14444 ns/iter reference)
import contextlib
import os

import jax
import jax.numpy as jnp
from jax import lax
from jax.experimental import pallas as pl
from jax.experimental.pallas import tpu as pltpu

N_DEV = 16
_PROFILE = bool(int(os.environ.get("KERNEL_PROFILE_SCOPES", "0")))


def _scope(name):
    return jax.named_scope(name) if _PROFILE else contextlib.nullcontext()


def kernel(x, w_mat):
    m_per, k = x.shape
    n = w_mat.shape[1]
    n_per = n // N_DEV

    def body(x_ref, w_ref, out_ref, yt_ref, r_ref, send_sems, recv_sems):
        me = lax.axis_index("i")

        barrier_sem = pltpu.get_barrier_semaphore()
        with _scope("barrier_signal"):
            for d in range(1, N_DEV):
                pl.semaphore_signal(
                    barrier_sem, inc=1,
                    device_id=(lax.rem(me + d, N_DEV),),
                    device_id_type=pl.DeviceIdType.MESH,
                )

        with _scope("gemm"):
            y_t = jnp.maximum(
                lax.dot_general(
                    w_ref[:, :], x_ref[:, :],
                    (((0,), (1,)), ((), ())),
                    preferred_element_type=jnp.float32,
                ),
                0.0,
            )
            yt_ref[:, :] = y_t

        with _scope("barrier_wait"):
            pl.semaphore_wait(barrier_sem, N_DEV - 1)

        with _scope("send_issue"):
            rdmas = []
            for d in range(1, N_DEV):
                j = lax.rem(me + d, N_DEV)
                rdma = pltpu.make_async_remote_copy(
                    src_ref=yt_ref.at[pl.ds(j * n_per, n_per), :],
                    dst_ref=r_ref.at[me],
                    send_sem=send_sems.at[d - 1],
                    recv_sem=recv_sems.at[d - 1],
                    device_id=(j,),
                    device_id_type=pl.DeviceIdType.MESH,
                )
                rdma.start()
                rdmas.append(rdma)

        with _scope("local_tile"):
            own = yt_ref[pl.ds(me * n_per, n_per), :]
            out_ref[pl.ds(me * m_per, m_per), :] = own.T

        with _scope("wait_recv"):
            for d in range(1, N_DEV):
                s = lax.rem(me + N_DEV - d, N_DEV)
                recv = pltpu.make_async_remote_copy(
                    src_ref=yt_ref.at[pl.ds(me * n_per, n_per), :],
                    dst_ref=r_ref.at[s],
                    send_sem=send_sems.at[d - 1],
                    recv_sem=recv_sems.at[d - 1],
                    device_id=(s,),
                    device_id_type=pl.DeviceIdType.MESH,
                )
                recv.wait_recv()
                out_ref[pl.ds(s * m_per, m_per), :] = r_ref[s].T

        with _scope("wait_send"):
            for rdma in rdmas:
                rdma.wait_send()

    out_shape = jax.ShapeDtypeStruct((N_DEV * m_per, n_per), jnp.float32)
    return pl.pallas_call(
        body,
        out_shape=out_shape,
        in_specs=[
            pl.BlockSpec(memory_space=pltpu.VMEM),
            pl.BlockSpec(memory_space=pltpu.VMEM),
        ],
        out_specs=pl.BlockSpec(memory_space=pltpu.VMEM),
        scratch_shapes=[
            pltpu.VMEM((n, m_per), jnp.float32),
            pltpu.VMEM((N_DEV, n_per, m_per), jnp.float32),
            pltpu.SemaphoreType.DMA((N_DEV - 1,)),
            pltpu.SemaphoreType.DMA((N_DEV - 1,)),
        ],
        compiler_params=pltpu.CompilerParams(collective_id=0),
    )(x, w_mat)


# device time: 14033 ns/iter; 1.0293x vs baseline; 1.0293x over previous
import contextlib
import os

import jax
import jax.numpy as jnp
from jax import lax
from jax.experimental import pallas as pl
from jax.experimental.pallas import tpu as pltpu

N_DEV = 16
_PROFILE = bool(int(os.environ.get("KERNEL_PROFILE_SCOPES", "0")))


def _scope(name):
    return jax.named_scope(name) if _PROFILE else contextlib.nullcontext()


def kernel(x, w_mat):
    m_per, k = x.shape
    n = w_mat.shape[1]
    n_per = n // N_DEV

    def body(x_ref, w_ref, out_ref, yt_ref, r_ref,
             send_sems, recv_sems, local_sem):
        me = lax.axis_index("i")

        barrier_sem = pltpu.get_barrier_semaphore()
        with _scope("barrier_signal"):
            for d in range(1, N_DEV):
                pl.semaphore_signal(
                    barrier_sem, inc=1,
                    device_id=(lax.rem(me + d, N_DEV),),
                    device_id_type=pl.DeviceIdType.MESH,
                )

        with _scope("gemm"):
            y_t = jnp.maximum(
                lax.dot_general(
                    w_ref[:, :], x_ref[:, :],
                    (((0,), (1,)), ((), ())),
                    preferred_element_type=jnp.float32,
                ),
                0.0,
            )
            yt_ref[:, :] = y_t

        with _scope("local_tile"):
            local = pltpu.make_async_copy(
                yt_ref.at[pl.ds(me * n_per, n_per), :],
                r_ref.at[me],
                local_sem,
            )
            local.start()

        with _scope("barrier_wait"):
            pl.semaphore_wait(barrier_sem, N_DEV - 1)

        with _scope("send_issue"):
            rdmas = []
            for d in range(1, N_DEV):
                j = lax.rem(me + d, N_DEV)
                rdma = pltpu.make_async_remote_copy(
                    src_ref=yt_ref.at[pl.ds(j * n_per, n_per), :],
                    dst_ref=r_ref.at[me],
                    send_sem=send_sems.at[d - 1],
                    recv_sem=recv_sems.at[d - 1],
                    device_id=(j,),
                    device_id_type=pl.DeviceIdType.MESH,
                )
                rdma.start()
                rdmas.append(rdma)

        with _scope("wait_send"):
            for rdma in rdmas:
                rdma.wait_send()
            local.wait()

        with _scope("wait_recv"):
            for d in range(1, N_DEV):
                s = lax.rem(me + N_DEV - d, N_DEV)
                recv = pltpu.make_async_remote_copy(
                    src_ref=yt_ref.at[pl.ds(me * n_per, n_per), :],
                    dst_ref=r_ref.at[s],
                    send_sem=send_sems.at[d - 1],
                    recv_sem=recv_sems.at[d - 1],
                    device_id=(s,),
                    device_id_type=pl.DeviceIdType.MESH,
                )
                recv.wait_recv()

        with _scope("untranspose"):
            for s in range(N_DEV):
                out_ref[s * m_per:(s + 1) * m_per, :] = r_ref[s].T

    out_shape = jax.ShapeDtypeStruct((N_DEV * m_per, n_per), jnp.float32)
    return pl.pallas_call(
        body,
        out_shape=out_shape,
        in_specs=[
            pl.BlockSpec(memory_space=pltpu.VMEM),
            pl.BlockSpec(memory_space=pltpu.VMEM),
        ],
        out_specs=pl.BlockSpec(memory_space=pltpu.VMEM),
        scratch_shapes=[
            pltpu.VMEM((n, m_per), jnp.float32),
            pltpu.VMEM((N_DEV, n_per, m_per), jnp.float32),
            pltpu.SemaphoreType.DMA((N_DEV - 1,)),
            pltpu.SemaphoreType.DMA((N_DEV - 1,)),
            pltpu.SemaphoreType.DMA,
        ],
        compiler_params=pltpu.CompilerParams(collective_id=0),
    )(x, w_mat)
